# Initial kernel scaffold; baseline (speedup 1.0000x reference)
import jax
import jax.numpy as jnp
from jax.experimental import pallas as pl


def kernel(x, edge_index, edge_attr, W_l, b_l, W_r, b_r, W_e, att, bias, W_fc, b_fc):
    raise NotImplementedError("write your pallas kernel here")


# speedup vs baseline: 3.9841x; 3.9841x over previous
import functools

import jax
import jax.numpy as jnp
from jax import lax
from jax.experimental import pallas as pl
from jax.experimental.pallas import tpu as pltpu
from jax.experimental.pallas import tpu_sc as plsc

N = 10000
F = 128
C = 256
CH = C // 2
O = 128
E = 320000
NPAD = 10240
NC, NS, L = 2, 16, 16
NW = NC * NS
EPW = E // NW
EPS = E // NS
K = 80
NCH1 = EPW // K
NCH2 = EPS // K
RPT = NPAD // NS
RB = 80

_mesh = plsc.VectorSubcoreMesh(
    core_axis_name="c", subcore_axis_name="s", num_cores=NC, num_subcores=NS)


def _hsum(v):
    lanes = lax.iota(jnp.int32, L)
    dnums = lax.GatherDimensionNumbers(
        offset_dims=(), collapsed_slice_dims=(0,), start_index_map=(0,))
    for sh in (1, 2, 4, 8):
        perm = lax.gather(
            v, (lanes ^ sh)[:, None], dnums, (1,),
            mode=lax.GatherScatterMode.PROMISE_IN_BOUNDS)
        v = v + perm
    return v


def _tc_proj_body(xp, wl, bl, wr, br, xla_o, xlb_o, xr_o):
    xl = jnp.dot(xp[...], wl[...], preferred_element_type=jnp.float32) + bl[...]
    xla_o[...] = xl[:, :CH]
    xlb_o[...] = xl[:, CH:]
    xr_o[...] = jnp.dot(xp[...], wr[...], preferred_element_type=jnp.float32) + br[...]


def _tc_maxcomb_body(locmax, mx_o):
    m = jnp.max(locmax[...], axis=0, keepdims=True)
    mx_o[...] = jnp.where(m > -1e37, m, 0.0)


def _tc_fc_body(ua, ub, biasa, biasb, wfa, wfb, bfc, out_o):
    t = jnp.dot(ua[...], wfa[...], preferred_element_type=jnp.float32)
    t = t + jnp.dot(ub[...], wfb[...], preferred_element_type=jnp.float32)
    const = (jnp.dot(biasa[...], wfa[...], preferred_element_type=jnp.float32)
             + jnp.dot(biasb[...], wfb[...], preferred_element_type=jnp.float32)
             + bfc[...])
    out_o[...] = t + const


def _sc_logits_body(xla, xlb, xr, src, dst, ea, we, att,
                    logits_o, locmax_o,
                    rows_a, rows_b, rows_r, srcv, dstv, eav, lv,
                    wev, attv, maxv, sem_a, sem_b, sem_r):
    wid = lax.axis_index("s") * NC + lax.axis_index("c")
    base = wid * EPW

    pltpu.sync_copy(we, wev)
    pltpu.sync_copy(att, attv)

    neg = jnp.full((L,), -jnp.inf, jnp.float32)

    @pl.loop(0, NPAD // L)
    def _(i):
        maxv[pl.ds(i * L, L)] = neg

    @pl.loop(0, NCH1)
    def _(i):
        off = pl.multiple_of(base + i * K, 8)
        pltpu.sync_copy(src.at[pl.ds(off, K)], srcv)
        pltpu.sync_copy(dst.at[pl.ds(off, K)], dstv)
        pltpu.sync_copy(ea.at[pl.ds(off, K)], eav)
        ca = pltpu.async_copy(xla.at[srcv], rows_a, sem_a)
        cb = pltpu.async_copy(xlb.at[srcv], rows_b, sem_b)
        cr = pltpu.async_copy(xr.at[dstv], rows_r, sem_r)
        ca.wait()
        cb.wait()
        cr.wait()

        lanes = lax.iota(jnp.int32, L)

        @pl.loop(0, K // L)
        def _(g):
            e16 = eav[pl.ds(g * L, L)]
            d16 = dstv[pl.ds(g * L, L)]
            lg16 = jnp.zeros((L,), jnp.float32)
            for t in range(L):
                k = g * L + t
                eak = e16[t]
                acc = jnp.zeros((L,), jnp.float32)
                for j in range(C // L):
                    if j < CH // L:
                        lpart = rows_a[k, pl.ds(j * L, L)]
                    else:
                        lpart = rows_b[k, pl.ds((j - CH // L) * L, L)]
                    m = (lpart + rows_r[k, pl.ds(j * L, L)]
                         + eak * wev[pl.ds(j * L, L)])
                    m = jnp.where(m > 0.0, m, 0.2 * m)
                    acc = acc + attv[pl.ds(j * L, L)] * m
                lgf = _hsum(acc)
                lg16 = jnp.where(lanes == t, lgf, lg16)
                db = jnp.full((L,), d16[t], jnp.int32)
                cur = plsc.load_gather(maxv, [db])
                plsc.store_scatter(maxv, [db], jnp.maximum(cur, lgf))
            lv[pl.ds(g * L, L)] = lg16

        pltpu.sync_copy(lv, logits_o.at[pl.ds(off, K)])

    pltpu.sync_copy(maxv, locmax_o.at[wid])


def _sc_accum_body(xla, xlb, src, dst, logits, mx,
                   ua_o, ub_o,
                   srcv, dstv, lvv, av, recv, rows, upd, mxv, dloc,
                   out_sh, den_sh, sem_g):
    c = lax.axis_index("c")
    s = lax.axis_index("s")
    rbase = s * RPT

    pltpu.sync_copy(mx, mxv)

    zv = jnp.zeros((L,), jnp.float32)

    @pl.loop(0, RB)
    def _(r):
        for j in range(CH // L):
            upd[r, pl.ds(j * L, L)] = zv

    @pl.loop(0, RPT // L)
    def _(i):
        dloc[pl.ds(i * L, L)] = zv

    for q in range(RPT // RB):
        pltpu.sync_copy(upd, out_sh.at[pl.ds(rbase + q * RB, RB)])
    pltpu.sync_copy(dloc, den_sh.at[pl.ds(rbase, RPT)])
    plsc.subcore_barrier()

    ebase = s * EPS

    @pl.loop(0, NCH2)
    def _(i):
        off = pl.multiple_of(ebase + i * K, 8)
        pltpu.sync_copy(src.at[pl.ds(off, K)], srcv)
        pltpu.sync_copy(dst.at[pl.ds(off, K)], dstv)
        pltpu.sync_copy(logits.at[pl.ds(off, K)], lvv)

        @pl.when(c == 0)
        def _():
            pltpu.async_copy(xla.at[srcv], rows, sem_g).wait()

        @pl.when(c == 1)
        def _():
            pltpu.async_copy(xlb.at[srcv], rows, sem_g).wait()

        @pl.loop(0, K // L)
        def _(g):
            d16 = dstv[pl.ds(g * L, L)]
            mxg = plsc.load_gather(mxv, [d16])
            a16 = jnp.exp(lvv[pl.ds(g * L, L)] - mxg)
            av[pl.ds(g * L, L)] = a16
            for t in range(L):
                k = g * L + t
                ak = a16[t]
                for j in range(CH // L):
                    upd[k, pl.ds(j * L, L)] = ak * rows[k, pl.ds(j * L, L)]

        pltpu.sync_copy(upd, out_sh.at[dstv], add=True)
        pltpu.sync_copy(av, den_sh.at[dstv], add=True)

    plsc.subcore_barrier()

    for q in range(RPT // RB):
        ro = rbase + q * RB
        pltpu.sync_copy(out_sh.at[pl.ds(ro, RB)], rows)
        pltpu.sync_copy(den_sh.at[pl.ds(ro, RB)], dloc.at[pl.ds(0, RB)])
        for g in range(RB // L):
            d16 = dloc[pl.ds(g * L, L)]
            r16 = 1.0 / (d16 + 1e-16)
            recv[pl.ds(g * L, L)] = r16
            for t in range(L):
                r = g * L + t
                rk = r16[t]
                for j in range(CH // L):
                    upd[r, pl.ds(j * L, L)] = rk * rows[r, pl.ds(j * L, L)]

        @pl.when(c == 0)
        def _():
            pltpu.sync_copy(upd, ua_o.at[pl.ds(ro, RB)])

        @pl.when(c == 1)
        def _():
            pltpu.sync_copy(upd, ub_o.at[pl.ds(ro, RB)])


_sc_logits = functools.partial(
    pl.kernel,
    out_type=(jax.ShapeDtypeStruct((E,), jnp.float32),
              jax.ShapeDtypeStruct((NW, NPAD), jnp.float32)),
    mesh=_mesh,
    compiler_params=pltpu.CompilerParams(needs_layout_passes=False),
    scratch_types=[
        pltpu.VMEM((K, CH), jnp.float32),
        pltpu.VMEM((K, CH), jnp.float32),
        pltpu.VMEM((K, C), jnp.float32),
        pltpu.VMEM((K,), jnp.int32),
        pltpu.VMEM((K,), jnp.int32),
        pltpu.VMEM((K,), jnp.float32),
        pltpu.VMEM((K,), jnp.float32),
        pltpu.VMEM((C,), jnp.float32),
        pltpu.VMEM((C,), jnp.float32),
        pltpu.VMEM((NPAD,), jnp.float32),
        pltpu.SemaphoreType.DMA,
        pltpu.SemaphoreType.DMA,
        pltpu.SemaphoreType.DMA,
    ],
)(_sc_logits_body)

_sc_accum = functools.partial(
    pl.kernel,
    out_type=(jax.ShapeDtypeStruct((NPAD, CH), jnp.float32),
              jax.ShapeDtypeStruct((NPAD, CH), jnp.float32)),
    mesh=_mesh,
    compiler_params=pltpu.CompilerParams(needs_layout_passes=False),
    scratch_types=[
        pltpu.VMEM((K,), jnp.int32),
        pltpu.VMEM((K,), jnp.int32),
        pltpu.VMEM((K,), jnp.float32),
        pltpu.VMEM((K,), jnp.float32),
        pltpu.VMEM((RB,), jnp.float32),
        pltpu.VMEM((RB, CH), jnp.float32),
        pltpu.VMEM((RB, CH), jnp.float32),
        pltpu.VMEM((NPAD,), jnp.float32),
        pltpu.VMEM((RPT,), jnp.float32),
        pltpu.VMEM_SHARED((NPAD, CH), jnp.float32),
        pltpu.VMEM_SHARED((NPAD,), jnp.float32),
        pltpu.SemaphoreType.DMA,
    ],
)(_sc_accum_body)


@jax.jit
def kernel(x, edge_index, edge_attr, W_l, b_l, W_r, b_r, W_e, att, bias,
           W_fc, b_fc):
    xs = x.reshape(N, F)
    xp = jnp.pad(xs, ((0, NPAD - N), (0, 0)))
    src = edge_index[0]
    dst = edge_index[1]
    ea = edge_attr.reshape(E)
    we = W_e.reshape(C)

    xla, xlb, xr = pl.pallas_call(
        _tc_proj_body,
        out_shape=(jax.ShapeDtypeStruct((NPAD, CH), jnp.float32),
                   jax.ShapeDtypeStruct((NPAD, CH), jnp.float32),
                   jax.ShapeDtypeStruct((NPAD, C), jnp.float32)),
    )(xp, W_l, b_l.reshape(1, C), W_r, b_r.reshape(1, C))

    logits, locmax = _sc_logits(xla, xlb, xr, src, dst, ea, we, att)

    mx = pl.pallas_call(
        _tc_maxcomb_body,
        out_shape=jax.ShapeDtypeStruct((1, NPAD), jnp.float32),
    )(locmax)

    ua, ub = _sc_accum(xla, xlb, src, dst, logits, mx.reshape(NPAD))

    out = pl.pallas_call(
        _tc_fc_body,
        out_shape=jax.ShapeDtypeStruct((NPAD, O), jnp.float32),
    )(ua, ub, bias[:CH].reshape(1, CH), bias[CH:].reshape(1, CH),
      W_fc[:CH], W_fc[CH:], b_fc.reshape(1, O))

    return out[:N].reshape(1, N, O)


# speedup vs baseline: 4.2125x; 1.0573x over previous
import functools

import jax
import jax.numpy as jnp
from jax import lax
from jax.experimental import pallas as pl
from jax.experimental.pallas import tpu as pltpu
from jax.experimental.pallas import tpu_sc as plsc

N = 10000
F = 128
C = 256
CH = C // 2
O = 128
E = 320000
NPAD = 10240
NC, NS, L = 2, 16, 16
NW = NC * NS
EPW = E // NW
EPS = E // NS
K = 80
NCH1 = EPW // K
NCH2 = EPS // K
RPT = NPAD // NS
RB = 80

_mesh = plsc.VectorSubcoreMesh(
    core_axis_name="c", subcore_axis_name="s", num_cores=NC, num_subcores=NS)


def _hsum(v):
    lanes = lax.iota(jnp.int32, L)
    dnums = lax.GatherDimensionNumbers(
        offset_dims=(), collapsed_slice_dims=(0,), start_index_map=(0,))
    for sh in (1, 2, 4, 8):
        perm = lax.gather(
            v, (lanes ^ sh)[:, None], dnums, (1,),
            mode=lax.GatherScatterMode.PROMISE_IN_BOUNDS)
        v = v + perm
    return v


def _tc_proj_body(xp, wl, bl, wr, br, xla_o, xlb_o, xr_o):
    xl = jnp.dot(xp[...], wl[...], preferred_element_type=jnp.float32) + bl[...]
    xla_o[...] = xl[:, :CH]
    xlb_o[...] = xl[:, CH:]
    xr_o[...] = jnp.dot(xp[...], wr[...], preferred_element_type=jnp.float32) + br[...]


def _tc_maxcomb_body(locmax, mx_o):
    m = jnp.max(locmax[...], axis=0, keepdims=True)
    mx_o[...] = jnp.where(m > -1e37, m, 0.0)


def _tc_fc_body(ua, ub, biasa, biasb, wfa, wfb, bfc, out_o):
    t = jnp.dot(ua[...], wfa[...], preferred_element_type=jnp.float32)
    t = t + jnp.dot(ub[...], wfb[...], preferred_element_type=jnp.float32)
    const = (jnp.dot(biasa[...], wfa[...], preferred_element_type=jnp.float32)
             + jnp.dot(biasb[...], wfb[...], preferred_element_type=jnp.float32)
             + bfc[...])
    out_o[...] = t + const


def _sc_logits_body(xla, xlb, xr, src, dst, ea, we, att,
                    logits_o, locmax_o,
                    rows_a, rows_b, rows_r, srcv, dstv, eav, lv,
                    wev, attv, maxv, sems):
    wid = lax.axis_index("s") * NC + lax.axis_index("c")
    base = wid * EPW

    pltpu.sync_copy(we, wev)
    pltpu.sync_copy(att, attv)

    neg = jnp.full((L,), -jnp.inf, jnp.float32)

    @pl.loop(0, NPAD // L)
    def _(i):
        maxv[pl.ds(i * L, L)] = neg

    def fetch(i, p):
        off = pl.multiple_of(base + i * K, 8)
        pltpu.sync_copy(src.at[pl.ds(off, K)], srcv[p])
        pltpu.sync_copy(dst.at[pl.ds(off, K)], dstv[p])
        pltpu.sync_copy(ea.at[pl.ds(off, K)], eav[p])
        pltpu.async_copy(xla.at[srcv[p]], rows_a[p], sems[p])
        pltpu.async_copy(xlb.at[srcv[p]], rows_b[p], sems[p])
        pltpu.async_copy(xr.at[dstv[p]], rows_r[p], sems[p])

    def consume(i, p):
        off = pl.multiple_of(base + i * K, 8)
        pltpu.make_async_copy(xla.at[srcv[p]], rows_a[p], sems[p]).wait()
        pltpu.make_async_copy(xlb.at[srcv[p]], rows_b[p], sems[p]).wait()
        pltpu.make_async_copy(xr.at[dstv[p]], rows_r[p], sems[p]).wait()
        lanes = lax.iota(jnp.int32, L)
        ra, rb, rr = rows_a[p], rows_b[p], rows_r[p]

        @pl.loop(0, K // L)
        def _(g):
            e16 = eav[p][pl.ds(g * L, L)]
            d16 = dstv[p][pl.ds(g * L, L)]
            lg16 = jnp.zeros((L,), jnp.float32)
            for t in range(L):
                k = g * L + t
                eak = e16[t]
                acc = jnp.zeros((L,), jnp.float32)
                for j in range(C // L):
                    if j < CH // L:
                        lpart = ra[k, pl.ds(j * L, L)]
                    else:
                        lpart = rb[k, pl.ds((j - CH // L) * L, L)]
                    m = (lpart + rr[k, pl.ds(j * L, L)]
                         + eak * wev[pl.ds(j * L, L)])
                    m = jnp.maximum(m, 0.2 * m)
                    acc = acc + attv[pl.ds(j * L, L)] * m
                lgf = _hsum(acc)
                lg16 = jnp.where(lanes == t, lgf, lg16)
                db = jnp.full((L,), d16[t], jnp.int32)
                cur = plsc.load_gather(maxv, [db])
                plsc.store_scatter(maxv, [db], jnp.maximum(cur, lgf))
            lv[pl.ds(g * L, L)] = lg16

        pltpu.sync_copy(lv, logits_o.at[pl.ds(off, K)])

    fetch(0, 0)

    @pl.loop(0, (NCH1 - 1) // 2)
    def _(h):
        fetch(2 * h + 1, 1)
        consume(2 * h, 0)
        fetch(2 * h + 2, 0)
        consume(2 * h + 1, 1)

    consume(NCH1 - 1, 0)

    pltpu.sync_copy(maxv, locmax_o.at[wid])


def _sc_accum_body(xla, xlb, src, dst, logits, mx,
                   ua_o, ub_o,
                   srcv, dstv, lvv, av, recv, rows, upd, mxv, dloc,
                   out_sh, den_sh, sems):
    c = lax.axis_index("c")
    s = lax.axis_index("s")
    rbase = s * RPT

    pltpu.sync_copy(mx, mxv)

    zv = jnp.zeros((L,), jnp.float32)

    @pl.loop(0, RB)
    def _(r):
        for j in range(CH // L):
            upd[r, pl.ds(j * L, L)] = zv

    @pl.loop(0, RPT // L)
    def _(i):
        dloc[pl.ds(i * L, L)] = zv

    for q in range(RPT // RB):
        pltpu.sync_copy(upd, out_sh.at[pl.ds(rbase + q * RB, RB)])
    pltpu.sync_copy(dloc, den_sh.at[pl.ds(rbase, RPT)])
    plsc.subcore_barrier()

    ebase = s * EPS

    def fetch(i, p):
        off = pl.multiple_of(ebase + i * K, 8)
        pltpu.sync_copy(src.at[pl.ds(off, K)], srcv[p])
        pltpu.sync_copy(dst.at[pl.ds(off, K)], dstv[p])
        pltpu.sync_copy(logits.at[pl.ds(off, K)], lvv[p])

        @pl.when(c == 0)
        def _():
            pltpu.async_copy(xla.at[srcv[p]], rows[p], sems[p])

        @pl.when(c == 1)
        def _():
            pltpu.async_copy(xlb.at[srcv[p]], rows[p], sems[p])

    def consume(i, p):
        pltpu.make_async_copy(xla.at[srcv[p]], rows[p], sems[p]).wait()
        rp = rows[p]

        @pl.loop(0, K // L)
        def _(g):
            d16 = dstv[p][pl.ds(g * L, L)]
            mxg = plsc.load_gather(mxv, [d16])
            a16 = jnp.exp(lvv[p][pl.ds(g * L, L)] - mxg)
            av[pl.ds(g * L, L)] = a16
            for t in range(L):
                k = g * L + t
                ak = a16[t]
                for j in range(CH // L):
                    upd[k, pl.ds(j * L, L)] = ak * rp[k, pl.ds(j * L, L)]

        pltpu.sync_copy(upd, out_sh.at[dstv[p]], add=True)
        pltpu.sync_copy(av, den_sh.at[dstv[p]], add=True)

    fetch(0, 0)

    @pl.loop(0, (NCH2 - 1) // 2)
    def _(h):
        fetch(2 * h + 1, 1)
        consume(2 * h, 0)
        fetch(2 * h + 2, 0)
        consume(2 * h + 1, 1)

    consume(NCH2 - 2, 0)
    fetch(NCH2 - 1, 1)
    consume(NCH2 - 1, 1)

    plsc.subcore_barrier()

    for q in range(RPT // RB):
        ro = rbase + q * RB
        pltpu.sync_copy(out_sh.at[pl.ds(ro, RB)], rows[0])
        pltpu.sync_copy(den_sh.at[pl.ds(ro, RB)], dloc.at[pl.ds(0, RB)])
        for g in range(RB // L):
            d16 = dloc[pl.ds(g * L, L)]
            r16 = 1.0 / (d16 + 1e-16)
            recv[pl.ds(g * L, L)] = r16
            for t in range(L):
                r = g * L + t
                rk = r16[t]
                for j in range(CH // L):
                    upd[r, pl.ds(j * L, L)] = rk * rows[0][r, pl.ds(j * L, L)]

        @pl.when(c == 0)
        def _():
            pltpu.sync_copy(upd, ua_o.at[pl.ds(ro, RB)])

        @pl.when(c == 1)
        def _():
            pltpu.sync_copy(upd, ub_o.at[pl.ds(ro, RB)])


_sc_logits = functools.partial(
    pl.kernel,
    out_type=(jax.ShapeDtypeStruct((E,), jnp.float32),
              jax.ShapeDtypeStruct((NW, NPAD), jnp.float32)),
    mesh=_mesh,
    compiler_params=pltpu.CompilerParams(needs_layout_passes=False),
    scratch_types=[
        (pltpu.VMEM((K, CH), jnp.float32), pltpu.VMEM((K, CH), jnp.float32)),
        (pltpu.VMEM((K, CH), jnp.float32), pltpu.VMEM((K, CH), jnp.float32)),
        (pltpu.VMEM((K, C), jnp.float32), pltpu.VMEM((K, C), jnp.float32)),
        (pltpu.VMEM((K,), jnp.int32), pltpu.VMEM((K,), jnp.int32)),
        (pltpu.VMEM((K,), jnp.int32), pltpu.VMEM((K,), jnp.int32)),
        (pltpu.VMEM((K,), jnp.float32), pltpu.VMEM((K,), jnp.float32)),
        pltpu.VMEM((K,), jnp.float32),
        pltpu.VMEM((C,), jnp.float32),
        pltpu.VMEM((C,), jnp.float32),
        pltpu.VMEM((NPAD,), jnp.float32),
        (pltpu.SemaphoreType.DMA, pltpu.SemaphoreType.DMA),
    ],
)(_sc_logits_body)

_sc_accum = functools.partial(
    pl.kernel,
    out_type=(jax.ShapeDtypeStruct((NPAD, CH), jnp.float32),
              jax.ShapeDtypeStruct((NPAD, CH), jnp.float32)),
    mesh=_mesh,
    compiler_params=pltpu.CompilerParams(needs_layout_passes=False),
    scratch_types=[
        (pltpu.VMEM((K,), jnp.int32), pltpu.VMEM((K,), jnp.int32)),
        (pltpu.VMEM((K,), jnp.int32), pltpu.VMEM((K,), jnp.int32)),
        (pltpu.VMEM((K,), jnp.float32), pltpu.VMEM((K,), jnp.float32)),
        pltpu.VMEM((K,), jnp.float32),
        pltpu.VMEM((RB,), jnp.float32),
        (pltpu.VMEM((K, CH), jnp.float32), pltpu.VMEM((K, CH), jnp.float32)),
        pltpu.VMEM((RB, CH), jnp.float32),
        pltpu.VMEM((NPAD,), jnp.float32),
        pltpu.VMEM((RPT,), jnp.float32),
        pltpu.VMEM_SHARED((NPAD, CH), jnp.float32),
        pltpu.VMEM_SHARED((NPAD,), jnp.float32),
        (pltpu.SemaphoreType.DMA, pltpu.SemaphoreType.DMA),
    ],
)(_sc_accum_body)


@jax.jit
def kernel(x, edge_index, edge_attr, W_l, b_l, W_r, b_r, W_e, att, bias,
           W_fc, b_fc):
    xs = x.reshape(N, F)
    xp = jnp.pad(xs, ((0, NPAD - N), (0, 0)))
    src = edge_index[0]
    dst = edge_index[1]
    ea = edge_attr.reshape(E)
    we = W_e.reshape(C)

    xla, xlb, xr = pl.pallas_call(
        _tc_proj_body,
        out_shape=(jax.ShapeDtypeStruct((NPAD, CH), jnp.float32),
                   jax.ShapeDtypeStruct((NPAD, CH), jnp.float32),
                   jax.ShapeDtypeStruct((NPAD, C), jnp.float32)),
    )(xp, W_l, b_l.reshape(1, C), W_r, b_r.reshape(1, C))

    logits, locmax = _sc_logits(xla, xlb, xr, src, dst, ea, we, att)

    mx = pl.pallas_call(
        _tc_maxcomb_body,
        out_shape=jax.ShapeDtypeStruct((1, NPAD), jnp.float32),
    )(locmax)

    ua, ub = _sc_accum(xla, xlb, src, dst, logits, mx.reshape(NPAD))

    out = pl.pallas_call(
        _tc_fc_body,
        out_shape=jax.ShapeDtypeStruct((NPAD, O), jnp.float32),
    )(ua, ub, bias[:CH].reshape(1, CH), bias[CH:].reshape(1, CH),
      W_fc[:CH], W_fc[CH:], b_fc.reshape(1, O))

    return out[:N].reshape(1, N, O)


# speedup vs baseline: 6.4111x; 1.5219x over previous
import functools

import jax
import jax.numpy as jnp
from jax import lax
from jax.experimental import pallas as pl
from jax.experimental.pallas import tpu as pltpu
from jax.experimental.pallas import tpu_sc as plsc

N = 10000
F = 128
C = 256
CH = C // 2
O = 128
E = 320000
NPAD = 10240
NC, NS, L = 2, 16, 16
NW = NC * NS
EPW = E // NW
EPS = E // NS
K = 80
NCH1 = EPW // K
NCH2 = EPS // K
RPT = NPAD // NS
RB = 80

_mesh = plsc.VectorSubcoreMesh(
    core_axis_name="c", subcore_axis_name="s", num_cores=NC, num_subcores=NS)


def _hsum(v):
    lanes = lax.iota(jnp.int32, L)
    dnums = lax.GatherDimensionNumbers(
        offset_dims=(), collapsed_slice_dims=(0,), start_index_map=(0,))
    for sh in (1, 2, 4, 8):
        perm = lax.gather(
            v, (lanes ^ sh)[:, None], dnums, (1,),
            mode=lax.GatherScatterMode.PROMISE_IN_BOUNDS)
        v = v + perm
    return v


def _tc_proj_body(xp, wl, bl, wr, br, xla_o, xlb_o, xr_o):
    xl = jnp.dot(xp[...], wl[...], preferred_element_type=jnp.float32) + bl[...]
    xla_o[...] = xl[:, :CH]
    xlb_o[...] = xl[:, CH:]
    xr_o[...] = jnp.dot(xp[...], wr[...], preferred_element_type=jnp.float32) + br[...]


def _tc_maxcomb_body(locmax, mx_o):
    m = jnp.max(locmax[...], axis=0, keepdims=True)
    mx_o[...] = jnp.where(m > -1e37, m, 0.0)


def _tc_fc_body(ua, ub, biasa, biasb, wfa, wfb, bfc, out_o):
    t = jnp.dot(ua[...], wfa[...], preferred_element_type=jnp.float32)
    t = t + jnp.dot(ub[...], wfb[...], preferred_element_type=jnp.float32)
    const = (jnp.dot(biasa[...], wfa[...], preferred_element_type=jnp.float32)
             + jnp.dot(biasb[...], wfb[...], preferred_element_type=jnp.float32)
             + bfc[...])
    out_o[...] = t + const


def _sc_logits_body(xla, xlb, xr, src, dst, ea, we, att,
                    logits_o, locmax_o,
                    rows_a, rows_b, rows_r, srcv, dstv, eav, lv,
                    wev, attv, maxv, sems):
    wid = lax.axis_index("s") * NC + lax.axis_index("c")
    base = wid * EPW

    pltpu.sync_copy(we, wev)
    pltpu.sync_copy(att, attv)

    neg = jnp.full((L,), -jnp.inf, jnp.float32)

    @pl.loop(0, NPAD // L)
    def _(i):
        maxv[pl.ds(i * L, L)] = neg

    def fetch(i, p):
        off = pl.multiple_of(base + i * K, 8)
        pltpu.sync_copy(src.at[pl.ds(off, K)], srcv[p])
        pltpu.sync_copy(dst.at[pl.ds(off, K)], dstv[p])
        pltpu.sync_copy(ea.at[pl.ds(off, K)], eav[p])
        pltpu.async_copy(xla.at[srcv[p]], rows_a[p], sems[p])
        pltpu.async_copy(xlb.at[srcv[p]], rows_b[p], sems[p])
        pltpu.async_copy(xr.at[dstv[p]], rows_r[p], sems[p])

    def consume(i, p):
        off = pl.multiple_of(base + i * K, 8)
        pltpu.make_async_copy(xla.at[srcv[p]], rows_a[p], sems[p]).wait()
        pltpu.make_async_copy(xlb.at[srcv[p]], rows_b[p], sems[p]).wait()
        pltpu.make_async_copy(xr.at[dstv[p]], rows_r[p], sems[p]).wait()
        ra, rb, rr = rows_a[p], rows_b[p], rows_r[p]

        @plsc.parallel_loop(0, K, unroll=2)
        def _(k):
            kb = jnp.full((L,), k, jnp.int32)
            ea16 = plsc.load_gather(eav[p], [kb])
            acc = jnp.zeros((L,), jnp.float32)
            for j in range(C // L):
                if j < CH // L:
                    lpart = ra[k, pl.ds(j * L, L)]
                else:
                    lpart = rb[k, pl.ds((j - CH // L) * L, L)]
                m = (lpart + rr[k, pl.ds(j * L, L)]
                     + ea16 * wev[pl.ds(j * L, L)])
                m = jnp.maximum(m, 0.2 * m)
                acc = acc + attv[pl.ds(j * L, L)] * m
            plsc.store_scatter(lv, [kb], _hsum(acc))

        @pl.loop(0, K)
        def _(k):
            kb = jnp.full((L,), k, jnp.int32)
            db = plsc.load_gather(dstv[p], [kb])
            lgb = plsc.load_gather(lv, [kb])
            cur = plsc.load_gather(maxv, [db])
            plsc.store_scatter(maxv, [db], jnp.maximum(cur, lgb))

        pltpu.sync_copy(lv, logits_o.at[pl.ds(off, K)])

    fetch(0, 0)

    @pl.loop(0, (NCH1 - 1) // 2)
    def _(h):
        fetch(2 * h + 1, 1)
        consume(2 * h, 0)
        fetch(2 * h + 2, 0)
        consume(2 * h + 1, 1)

    consume(NCH1 - 1, 0)

    pltpu.sync_copy(maxv, locmax_o.at[wid])


def _sc_accum_body(xla, xlb, src, dst, logits, mx,
                   ua_o, ub_o,
                   srcv, dstv, lvv, av, recv, rows, upd, mxv, dloc,
                   out_sh, den_sh, sems):
    c = lax.axis_index("c")
    s = lax.axis_index("s")
    rbase = s * RPT

    pltpu.sync_copy(mx, mxv)

    zv = jnp.zeros((L,), jnp.float32)

    @pl.loop(0, RB)
    def _(r):
        for j in range(CH // L):
            upd[r, pl.ds(j * L, L)] = zv

    @pl.loop(0, RPT // L)
    def _(i):
        dloc[pl.ds(i * L, L)] = zv

    for q in range(RPT // RB):
        pltpu.sync_copy(upd, out_sh.at[pl.ds(rbase + q * RB, RB)])
    pltpu.sync_copy(dloc, den_sh.at[pl.ds(rbase, RPT)])
    plsc.subcore_barrier()

    ebase = s * EPS

    def fetch(i, p):
        off = pl.multiple_of(ebase + i * K, 8)
        pltpu.sync_copy(src.at[pl.ds(off, K)], srcv[p])
        pltpu.sync_copy(dst.at[pl.ds(off, K)], dstv[p])
        pltpu.sync_copy(logits.at[pl.ds(off, K)], lvv[p])

        @pl.when(c == 0)
        def _():
            pltpu.async_copy(xla.at[srcv[p]], rows[p], sems[p])

        @pl.when(c == 1)
        def _():
            pltpu.async_copy(xlb.at[srcv[p]], rows[p], sems[p])

    def consume(i, p):
        pltpu.make_async_copy(xla.at[srcv[p]], rows[p], sems[p]).wait()
        rp = rows[p]

        @plsc.parallel_loop(0, K // L)
        def _(g):
            d16 = dstv[p][pl.ds(g * L, L)]
            mxg = plsc.load_gather(mxv, [d16])
            a16 = jnp.exp(lvv[p][pl.ds(g * L, L)] - mxg)
            av[pl.ds(g * L, L)] = a16
            for t in range(L):
                k = g * L + t
                ak = a16[t]
                for j in range(CH // L):
                    upd[k, pl.ds(j * L, L)] = ak * rp[k, pl.ds(j * L, L)]

        pltpu.sync_copy(upd, out_sh.at[dstv[p]], add=True)
        pltpu.sync_copy(av, den_sh.at[dstv[p]], add=True)

    fetch(0, 0)

    @pl.loop(0, (NCH2 - 1) // 2)
    def _(h):
        fetch(2 * h + 1, 1)
        consume(2 * h, 0)
        fetch(2 * h + 2, 0)
        consume(2 * h + 1, 1)

    consume(NCH2 - 2, 0)
    fetch(NCH2 - 1, 1)
    consume(NCH2 - 1, 1)

    plsc.subcore_barrier()

    for q in range(RPT // RB):
        ro = rbase + q * RB
        pltpu.sync_copy(out_sh.at[pl.ds(ro, RB)], rows[0])
        pltpu.sync_copy(den_sh.at[pl.ds(ro, RB)], dloc.at[pl.ds(0, RB)])
        for g in range(RB // L):
            d16 = dloc[pl.ds(g * L, L)]
            r16 = 1.0 / (d16 + 1e-16)
            recv[pl.ds(g * L, L)] = r16
            for t in range(L):
                r = g * L + t
                rk = r16[t]
                for j in range(CH // L):
                    upd[r, pl.ds(j * L, L)] = rk * rows[0][r, pl.ds(j * L, L)]

        @pl.when(c == 0)
        def _():
            pltpu.sync_copy(upd, ua_o.at[pl.ds(ro, RB)])

        @pl.when(c == 1)
        def _():
            pltpu.sync_copy(upd, ub_o.at[pl.ds(ro, RB)])


_sc_logits = functools.partial(
    pl.kernel,
    out_type=(jax.ShapeDtypeStruct((E,), jnp.float32),
              jax.ShapeDtypeStruct((NW, NPAD), jnp.float32)),
    mesh=_mesh,
    compiler_params=pltpu.CompilerParams(needs_layout_passes=False),
    scratch_types=[
        (pltpu.VMEM((K, CH), jnp.float32), pltpu.VMEM((K, CH), jnp.float32)),
        (pltpu.VMEM((K, CH), jnp.float32), pltpu.VMEM((K, CH), jnp.float32)),
        (pltpu.VMEM((K, C), jnp.float32), pltpu.VMEM((K, C), jnp.float32)),
        (pltpu.VMEM((K,), jnp.int32), pltpu.VMEM((K,), jnp.int32)),
        (pltpu.VMEM((K,), jnp.int32), pltpu.VMEM((K,), jnp.int32)),
        (pltpu.VMEM((K,), jnp.float32), pltpu.VMEM((K,), jnp.float32)),
        pltpu.VMEM((K,), jnp.float32),
        pltpu.VMEM((C,), jnp.float32),
        pltpu.VMEM((C,), jnp.float32),
        pltpu.VMEM((NPAD,), jnp.float32),
        (pltpu.SemaphoreType.DMA, pltpu.SemaphoreType.DMA),
    ],
)(_sc_logits_body)

_sc_accum = functools.partial(
    pl.kernel,
    out_type=(jax.ShapeDtypeStruct((NPAD, CH), jnp.float32),
              jax.ShapeDtypeStruct((NPAD, CH), jnp.float32)),
    mesh=_mesh,
    compiler_params=pltpu.CompilerParams(needs_layout_passes=False),
    scratch_types=[
        (pltpu.VMEM((K,), jnp.int32), pltpu.VMEM((K,), jnp.int32)),
        (pltpu.VMEM((K,), jnp.int32), pltpu.VMEM((K,), jnp.int32)),
        (pltpu.VMEM((K,), jnp.float32), pltpu.VMEM((K,), jnp.float32)),
        pltpu.VMEM((K,), jnp.float32),
        pltpu.VMEM((RB,), jnp.float32),
        (pltpu.VMEM((K, CH), jnp.float32), pltpu.VMEM((K, CH), jnp.float32)),
        pltpu.VMEM((RB, CH), jnp.float32),
        pltpu.VMEM((NPAD,), jnp.float32),
        pltpu.VMEM((RPT,), jnp.float32),
        pltpu.VMEM_SHARED((NPAD, CH), jnp.float32),
        pltpu.VMEM_SHARED((NPAD,), jnp.float32),
        (pltpu.SemaphoreType.DMA, pltpu.SemaphoreType.DMA),
    ],
)(_sc_accum_body)


@jax.jit
def kernel(x, edge_index, edge_attr, W_l, b_l, W_r, b_r, W_e, att, bias,
           W_fc, b_fc):
    xs = x.reshape(N, F)
    xp = jnp.pad(xs, ((0, NPAD - N), (0, 0)))
    src = edge_index[0]
    dst = edge_index[1]
    ea = edge_attr.reshape(E)
    we = W_e.reshape(C)

    xla, xlb, xr = pl.pallas_call(
        _tc_proj_body,
        out_shape=(jax.ShapeDtypeStruct((NPAD, CH), jnp.float32),
                   jax.ShapeDtypeStruct((NPAD, CH), jnp.float32),
                   jax.ShapeDtypeStruct((NPAD, C), jnp.float32)),
    )(xp, W_l, b_l.reshape(1, C), W_r, b_r.reshape(1, C))

    logits, locmax = _sc_logits(xla, xlb, xr, src, dst, ea, we, att)

    mx = pl.pallas_call(
        _tc_maxcomb_body,
        out_shape=jax.ShapeDtypeStruct((1, NPAD), jnp.float32),
    )(locmax)

    ua, ub = _sc_accum(xla, xlb, src, dst, logits, mx.reshape(NPAD))

    out = pl.pallas_call(
        _tc_fc_body,
        out_shape=jax.ShapeDtypeStruct((NPAD, O), jnp.float32),
    )(ua, ub, bias[:CH].reshape(1, CH), bias[CH:].reshape(1, CH),
      W_fc[:CH], W_fc[CH:], b_fc.reshape(1, O))

    return out[:N].reshape(1, N, O)
